# Initial kernel scaffold; baseline (speedup 1.0000x reference)
import jax
import jax.numpy as jnp
from jax.experimental import pallas as pl


def kernel(boxes, scores, idxs):
    raise NotImplementedError("write your pallas kernel here")


# speedup vs baseline: 217.0233x; 217.0233x over previous
import functools

import jax
import jax.numpy as jnp
from jax import lax
from jax.experimental import pallas as pl
from jax.experimental.pallas import tpu as pltpu
from jax.experimental.pallas import tpu_sc as plsc

_INP = 416.0
_OFF = 418.0
_THRES = 0.3
_K = 8
_L = 16
_NPAD = 5120
_NSL = _NPAD // _L


def _nms_body(cx_h, cy_h, w_h, h_h, sc_h, cl_h, out_h,
              cx_v, cy_v, w_v, h_v, sc_v, cl_v,
              x1_v, y1_v, x2_v, y2_v, ar_v, out_v):
    @pl.when((lax.axis_index("c") == 0) & (lax.axis_index("s") == 0))
    def _():
        pltpu.sync_copy(cx_h, cx_v)
        pltpu.sync_copy(cy_h, cy_v)
        pltpu.sync_copy(w_h, w_v)
        pltpu.sync_copy(h_h, h_v)
        pltpu.sync_copy(sc_h, sc_v)
        pltpu.sync_copy(cl_h, cl_v)

        lane = lax.iota(jnp.int32, _L)

        def conv(i, carry):
            sl = pl.ds(i * _L, _L)
            cx = cx_v[sl]
            cy = cy_v[sl]
            w = w_v[sl]
            h = h_v[sl]
            off = cl_v[sl].astype(jnp.float32) * _OFF
            x1 = jnp.minimum(jnp.maximum(cx - w / 2.0, 0.0), _INP) + off
            y1 = jnp.minimum(jnp.maximum(cy - h / 2.0, 0.0), _INP) + off
            x2 = jnp.minimum(jnp.maximum(cx + w / 2.0, 0.0), _INP) + off
            y2 = jnp.minimum(jnp.maximum(cy + h / 2.0, 0.0), _INP) + off
            x1_v[sl] = x1
            y1_v[sl] = y1
            x2_v[sl] = x2
            y2_v[sl] = y2
            ar_v[sl] = (x2 - x1 + 1.0) * (y2 - y1 + 1.0)
            return carry

        lax.fori_loop(0, _NSL, conv, 0)

        zero_i = jnp.zeros((_L,), jnp.int32)

        for k in range(_K):
            def amax(i, carry):
                bv, bi = carry
                v = sc_v[pl.ds(i * _L, _L)]
                idx = lane + i * _L
                upd = v > bv
                return jnp.where(upd, v, bv), jnp.where(upd, idx, bi)

            bv, bi = lax.fori_loop(
                0, _NSL, amax,
                (jnp.full((_L,), -2.0, jnp.float32), zero_i),
            )
            for s in (1, 2, 4, 8):
                p = lane ^ s
                bv2 = bv.at[p].get(mode="promise_in_bounds")
                bi2 = bi.at[p].get(mode="promise_in_bounds")
                take = (bv2 > bv) | ((bv2 == bv) & (bi2 < bi))
                bv = jnp.where(take, bv2, bv)
                bi = jnp.where(take, bi2, bi)
            mv = bv
            giv = bi

            px1 = plsc.load_gather(x1_v, [giv])
            py1 = plsc.load_gather(y1_v, [giv])
            px2 = plsc.load_gather(x2_v, [giv])
            py2 = plsc.load_gather(y2_v, [giv])
            pa = plsc.load_gather(ar_v, [giv])

            pcx = plsc.load_gather(cx_v, [giv])
            pcy = plsc.load_gather(cy_v, [giv])
            pw = plsc.load_gather(w_v, [giv])
            ph = plsc.load_gather(h_v, [giv])
            pcl = plsc.load_gather(cl_v, [giv]).astype(jnp.float32)
            ux1 = jnp.minimum(jnp.maximum(pcx - pw / 2.0, 0.0), _INP)
            uy1 = jnp.minimum(jnp.maximum(pcy - ph / 2.0, 0.0), _INP)
            ux2 = jnp.minimum(jnp.maximum(pcx + pw / 2.0, 0.0), _INP)
            uy2 = jnp.minimum(jnp.maximum(pcy + ph / 2.0, 0.0), _INP)
            row = jnp.where(lane == 0, ux1, 0.0)
            row = jnp.where(lane == 1, uy1, row)
            row = jnp.where(lane == 2, ux2, row)
            row = jnp.where(lane == 3, uy2, row)
            row = jnp.where(lane == 4, mv, row)
            row = jnp.where(lane == 5, pcl, row)
            out_v[pl.ds(k * _L, _L)] = row

            def sup(i, carry):
                sl = pl.ds(i * _L, _L)
                x1 = x1_v[sl]
                y1 = y1_v[sl]
                x2 = x2_v[sl]
                y2 = y2_v[sl]
                a = ar_v[sl]
                ix1 = jnp.maximum(px1, x1)
                iy1 = jnp.maximum(py1, y1)
                ix2 = jnp.minimum(px2, x2)
                iy2 = jnp.minimum(py2, y2)
                inter = (jnp.maximum(ix2 - ix1 + 1.0, 0.0)
                         * jnp.maximum(iy2 - iy1 + 1.0, 0.0))
                iou = inter / (pa + a - inter + 1e-16)
                sc_v[sl] = jnp.where(iou > _THRES, -1.0, sc_v[sl])
                return carry

            lax.fori_loop(0, _NSL, sup, 0)

        pltpu.sync_copy(out_v, out_h)


_nms_sc = functools.partial(
    pl.kernel,
    out_type=jax.ShapeDtypeStruct((_K * _L,), jnp.float32),
    mesh=plsc.VectorSubcoreMesh(core_axis_name="c", subcore_axis_name="s"),
    compiler_params=pltpu.CompilerParams(needs_layout_passes=False),
    scratch_types=[
        pltpu.VMEM((_NPAD,), jnp.float32),
        pltpu.VMEM((_NPAD,), jnp.float32),
        pltpu.VMEM((_NPAD,), jnp.float32),
        pltpu.VMEM((_NPAD,), jnp.float32),
        pltpu.VMEM((_NPAD,), jnp.float32),
        pltpu.VMEM((_NPAD,), jnp.int32),
        pltpu.VMEM((_NPAD,), jnp.float32),
        pltpu.VMEM((_NPAD,), jnp.float32),
        pltpu.VMEM((_NPAD,), jnp.float32),
        pltpu.VMEM((_NPAD,), jnp.float32),
        pltpu.VMEM((_NPAD,), jnp.float32),
        pltpu.VMEM((_K * _L,), jnp.float32),
    ],
)(_nms_body)


def kernel(boxes, scores, idxs):
    n = boxes.shape[0]
    bp = jnp.zeros((_NPAD, 4), jnp.float32).at[:n].set(boxes)
    sp = jnp.full((_NPAD,), -1.0, jnp.float32).at[:n].set(scores)
    cp = jnp.zeros((_NPAD,), jnp.int32).at[:n].set(idxs)
    out = _nms_sc(bp[:, 0], bp[:, 1], bp[:, 2], bp[:, 3], sp, cp)
    return out.reshape(_K, _L)[:, :6]


# speedup vs baseline: 299.6228x; 1.3806x over previous
import functools

import jax
import jax.numpy as jnp
from jax import lax
from jax.experimental import pallas as pl
from jax.experimental.pallas import tpu as pltpu
from jax.experimental.pallas import tpu_sc as plsc

_INP = 416.0
_OFF = 418.0
_THRES = 0.3
_K = 8
_L = 16
_NPAD = 5120
_NSL = _NPAD // _L


def _nms_body(cx_h, cy_h, w_h, h_h, sc_h, cl_h, out_h,
              cx_v, cy_v, w_v, h_v, sc_v, cl_v,
              x1_v, y1_v, x2_v, y2_v, ar_v, out_v):
    @pl.when((lax.axis_index("c") == 0) & (lax.axis_index("s") == 0))
    def _():
        pltpu.sync_copy(cx_h, cx_v)
        pltpu.sync_copy(cy_h, cy_v)
        pltpu.sync_copy(w_h, w_v)
        pltpu.sync_copy(h_h, h_v)
        pltpu.sync_copy(sc_h, sc_v)
        pltpu.sync_copy(cl_h, cl_v)

        lane = lax.iota(jnp.int32, _L)
        neg2 = jnp.full((_L,), -2.0, jnp.float32)
        zero_i = jnp.zeros((_L,), jnp.int32)

        def conv_amax(i, carry):
            bv, bi = carry
            for u in range(2):
                j = 2 * i + u
                sl = pl.ds(j * _L, _L)
                cx = cx_v[sl]
                cy = cy_v[sl]
                w = w_v[sl]
                h = h_v[sl]
                off = cl_v[sl].astype(jnp.float32) * _OFF
                x1 = jnp.minimum(jnp.maximum(cx - w / 2.0, 0.0), _INP) + off
                y1 = jnp.minimum(jnp.maximum(cy - h / 2.0, 0.0), _INP) + off
                x2 = jnp.minimum(jnp.maximum(cx + w / 2.0, 0.0), _INP) + off
                y2 = jnp.minimum(jnp.maximum(cy + h / 2.0, 0.0), _INP) + off
                x1_v[sl] = x1
                y1_v[sl] = y1
                x2_v[sl] = x2
                y2_v[sl] = y2
                ar_v[sl] = (x2 - x1 + 1.0) * (y2 - y1 + 1.0)
                s = sc_v[sl]
                upd = s > bv
                bv = jnp.where(upd, s, bv)
                bi = jnp.where(upd, lane + j * _L, bi)
            return bv, bi

        bv, bi = lax.fori_loop(0, _NSL // 2, conv_amax, (neg2, zero_i))

        for k in range(_K):
            for s in (1, 2, 4, 8):
                p = lane ^ s
                bv2 = bv.at[p].get(mode="promise_in_bounds")
                bi2 = bi.at[p].get(mode="promise_in_bounds")
                take = (bv2 > bv) | ((bv2 == bv) & (bi2 < bi))
                bv = jnp.where(take, bv2, bv)
                bi = jnp.where(take, bi2, bi)
            mv = bv
            giv = bi

            px1 = plsc.load_gather(x1_v, [giv])
            py1 = plsc.load_gather(y1_v, [giv])
            px2 = plsc.load_gather(x2_v, [giv])
            py2 = plsc.load_gather(y2_v, [giv])
            pa = plsc.load_gather(ar_v, [giv])

            pcx = plsc.load_gather(cx_v, [giv])
            pcy = plsc.load_gather(cy_v, [giv])
            pw = plsc.load_gather(w_v, [giv])
            ph = plsc.load_gather(h_v, [giv])
            pcl = plsc.load_gather(cl_v, [giv]).astype(jnp.float32)
            ux1 = jnp.minimum(jnp.maximum(pcx - pw / 2.0, 0.0), _INP)
            uy1 = jnp.minimum(jnp.maximum(pcy - ph / 2.0, 0.0), _INP)
            ux2 = jnp.minimum(jnp.maximum(pcx + pw / 2.0, 0.0), _INP)
            uy2 = jnp.minimum(jnp.maximum(pcy + ph / 2.0, 0.0), _INP)
            row = jnp.where(lane == 0, ux1, 0.0)
            row = jnp.where(lane == 1, uy1, row)
            row = jnp.where(lane == 2, ux2, row)
            row = jnp.where(lane == 3, uy2, row)
            row = jnp.where(lane == 4, mv, row)
            row = jnp.where(lane == 5, pcl, row)
            out_v[pl.ds(k * _L, _L)] = row

            if k == _K - 1:
                break

            def sup_amax(i, carry):
                bv, bi = carry
                for u in range(2):
                    j = 2 * i + u
                    sl = pl.ds(j * _L, _L)
                    x1 = x1_v[sl]
                    y1 = y1_v[sl]
                    x2 = x2_v[sl]
                    y2 = y2_v[sl]
                    a = ar_v[sl]
                    ix1 = jnp.maximum(px1, x1)
                    iy1 = jnp.maximum(py1, y1)
                    ix2 = jnp.minimum(px2, x2)
                    iy2 = jnp.minimum(py2, y2)
                    inter = (jnp.maximum(ix2 - ix1 + 1.0, 0.0)
                             * jnp.maximum(iy2 - iy1 + 1.0, 0.0))
                    iou = inter / (pa + a - inter + 1e-16)
                    s = jnp.where(iou > _THRES, -1.0, sc_v[sl])
                    sc_v[sl] = s
                    upd = s > bv
                    bv = jnp.where(upd, s, bv)
                    bi = jnp.where(upd, lane + j * _L, bi)
                return bv, bi

            bv, bi = lax.fori_loop(0, _NSL // 2, sup_amax, (neg2, zero_i))

        pltpu.sync_copy(out_v, out_h)


_nms_sc = functools.partial(
    pl.kernel,
    out_type=jax.ShapeDtypeStruct((_K * _L,), jnp.float32),
    mesh=plsc.VectorSubcoreMesh(core_axis_name="c", subcore_axis_name="s"),
    compiler_params=pltpu.CompilerParams(needs_layout_passes=False),
    scratch_types=[
        pltpu.VMEM((_NPAD,), jnp.float32),
        pltpu.VMEM((_NPAD,), jnp.float32),
        pltpu.VMEM((_NPAD,), jnp.float32),
        pltpu.VMEM((_NPAD,), jnp.float32),
        pltpu.VMEM((_NPAD,), jnp.float32),
        pltpu.VMEM((_NPAD,), jnp.int32),
        pltpu.VMEM((_NPAD,), jnp.float32),
        pltpu.VMEM((_NPAD,), jnp.float32),
        pltpu.VMEM((_NPAD,), jnp.float32),
        pltpu.VMEM((_NPAD,), jnp.float32),
        pltpu.VMEM((_NPAD,), jnp.float32),
        pltpu.VMEM((_K * _L,), jnp.float32),
    ],
)(_nms_body)


def kernel(boxes, scores, idxs):
    n = boxes.shape[0]
    bp = jnp.zeros((_NPAD, 4), jnp.float32).at[:n].set(boxes)
    sp = jnp.full((_NPAD,), -1.0, jnp.float32).at[:n].set(scores)
    cp = jnp.zeros((_NPAD,), jnp.int32).at[:n].set(idxs)
    out = _nms_sc(bp[:, 0], bp[:, 1], bp[:, 2], bp[:, 3], sp, cp)
    return out.reshape(_K, _L)[:, :6]
